# baseline (speedup 1.0000x reference)
import jax
import jax.numpy as jnp
from jax import lax
from jax.experimental import pallas as pl
from jax.experimental.pallas import tpu as pltpu
from jax.experimental.pallas import tpu_sc as plsc

BATCH = 4096
HIST = 200
EMBED_DIM = 64
V = 1_000_000

NC = 2
NS = 16
NW = NC * NS
ROWS_W = BATCH // NW

NBLK = V // 128
KPW = NBLK // NW
NLEFT = NBLK - KPW * NW
RAG_COLS = V - NBLK * 128
FROWS = V * EMBED_DIM // 128


def _iota16():
    return lax.iota(jnp.int32, 16)


def _transpose_block(src, dst2, b):
    bvec = jnp.full((16,), b, jnp.int32)
    for c in range(EMBED_DIM // 16):
        val = plsc.load_gather(src, [_iota16() + c * 16, bvec])
        p = b * EMBED_DIM + c * 16
        dst2[p // 128, pl.ds(p % 128, 16)] = val


def _gather_body(words_hbm, tbl_hbm, out5_hbm, idx_v, idx_t, rows2, rt2,
                 gsems, osems):
    wid = lax.axis_index("s") * NC + lax.axis_index("c")
    base = wid * ROWS_W
    pltpu.sync_copy(words_hbm.at[pl.ds(base, ROWS_W)], idx_v)

    def tr_idx(hh, _):
        hvec = jnp.full((16,), hh, jnp.int32)
        for c in range(ROWS_W // 16):
            val = plsc.load_gather(idx_v, [_iota16() + c * 16, hvec])
            idx_t[hh, pl.ds(c * 16, 16)] = val
        return ()

    lax.fori_loop(0, HIST, tr_idx, (), unroll=False)

    def fire_gather(i, h):
        pltpu.async_copy(tbl_hbm.at[idx_t.at[i]], rows2.at[h], gsems[h])

    def transpose_rows(h):
        for e in range(EMBED_DIM):
            evec = jnp.full((16,), e, jnp.int32)
            for c in range(ROWS_W // 16):
                val = plsc.load_gather(rows2.at[h], [_iota16() + c * 16, evec])
                rt2[h, e // 8, e % 8, pl.ds(c * 16, 16)] = val

    def body(i, _):
        h = jnp.remainder(i, 2)
        for hh in (0, 1):
            @pl.when(h == hh)
            def _():
                @pl.when(i + 1 < HIST)
                def _():
                    fire_gather(i + 1, 1 - hh)

                pltpu.make_async_copy(
                    tbl_hbm.at[pl.ds(0, ROWS_W)], rows2.at[hh], gsems[hh]
                ).wait()

                @pl.when(i >= 2)
                def _():
                    pltpu.make_async_copy(
                        rt2.at[hh], out5_hbm.at[0, :, 0], osems[hh]
                    ).wait()

                transpose_rows(hh)
                pltpu.async_copy(rt2.at[hh], out5_hbm.at[i, :, wid], osems[hh])
        return ()

    fire_gather(0, 0)
    lax.fori_loop(0, HIST, body, (), unroll=False)
    for hh in (0, 1):
        pltpu.make_async_copy(
            rt2.at[hh], out5_hbm.at[0, :, 0], osems[hh]
        ).wait()


def kernel(words, table):
    mesh = plsc.VectorSubcoreMesh(core_axis_name="c", subcore_axis_name="s")
    out5 = pl.kernel(
        _gather_body,
        out_type=jax.ShapeDtypeStruct((HIST, 8, NW, 8, 128), jnp.float32),
        mesh=mesh,
        scratch_types=[
            pltpu.VMEM((ROWS_W, HIST), jnp.int32),
            pltpu.VMEM((HIST, ROWS_W), jnp.int32),
            pltpu.VMEM((2, ROWS_W, EMBED_DIM), jnp.float32),
            pltpu.VMEM((2, 8, 8, 128), jnp.float32),
            [pltpu.SemaphoreType.DMA, pltpu.SemaphoreType.DMA],
            [pltpu.SemaphoreType.DMA, pltpu.SemaphoreType.DMA],
        ],
        compiler_params=pltpu.CompilerParams(use_tc_tiling_on_sc=False, needs_layout_passes=False),
    )(words, table)
    return out5.transpose(2, 4, 0, 1, 3).reshape(BATCH, HIST, EMBED_DIM)


# speedup vs baseline: 2.0797x; 2.0797x over previous
import jax
import jax.numpy as jnp
from jax import lax
from jax.experimental import pallas as pl
from jax.experimental.pallas import tpu as pltpu
from jax.experimental.pallas import tpu_sc as plsc

BATCH = 4096
HIST = 200
EMBED_DIM = 64
V = 1_000_000

NC = 2
NS = 16
NW = NC * NS
ROWS_W = BATCH // NW

NBLK = V // 128
KPW = NBLK // NW
NLEFT = NBLK - KPW * NW
RAG_COLS = V - NBLK * 128
RPAD = 69


def _iota16():
    return lax.iota(jnp.int32, 16)


def _transpose_block(src, dst2, b):
    bvec = jnp.full((16,), b, jnp.int32)
    for c in range(EMBED_DIM // 16):
        val = plsc.load_gather(src, [_iota16() + c * 16, bvec])
        p = b * EMBED_DIM + c * 16
        dst2[p // 128, pl.ds(p % 128, 16)] = val


def _gather_body(words_hbm, tbl_hbm, out5_hbm, idx_v, idx_t, rows2, rt2,
                 gsems, osems):
    wid = lax.axis_index("s") * NC + lax.axis_index("c")
    base = wid * ROWS_W
    pltpu.sync_copy(words_hbm.at[pl.ds(base, ROWS_W)], idx_v)

    def tr_idx(hh, _):
        hvec = jnp.full((16,), hh, jnp.int32)
        for c in range(ROWS_W // 16):
            val = plsc.load_gather(idx_v, [_iota16() + c * 16, hvec])
            idx_t[hh, pl.ds(c * 16, 16)] = val
        return ()

    lax.fori_loop(0, HIST, tr_idx, (), unroll=False)

    def fire_gather(i, h):
        pltpu.async_copy(tbl_hbm.at[idx_t.at[i]], rows2.at[h], gsems[h])

    def transpose_rows(h):
        def step(e0, _):
            rot = jnp.remainder(_iota16() + e0, 16)
            for e1 in range(EMBED_DIM // 16):
                ev = e1 * 16 + rot
                i1 = ev // 8
                i2 = jnp.remainder(ev, 8)
                for b0 in range(0, ROWS_W, 16):
                    bvec = _iota16() + b0
                    val = plsc.load_gather(rows2.at[h], [bvec, ev])
                    plsc.store_scatter(rt2.at[h], [i1, i2, bvec], val)
            return ()

        lax.fori_loop(0, 16, step, (), unroll=False)

    def body(i, _):
        h = jnp.remainder(i, 2)
        for hh in (0, 1):
            @pl.when(h == hh)
            def _():
                @pl.when(i + 1 < HIST)
                def _():
                    fire_gather(i + 1, 1 - hh)

                pltpu.make_async_copy(
                    tbl_hbm.at[pl.ds(0, ROWS_W)], rows2.at[hh], gsems[hh]
                ).wait()

                @pl.when(i >= 2)
                def _():
                    pltpu.make_async_copy(
                        rt2.at[hh], out5_hbm.at[0, :, 0], osems[hh]
                    ).wait()

                transpose_rows(hh)
                pltpu.async_copy(rt2.at[hh], out5_hbm.at[i, :, wid], osems[hh])
        return ()

    fire_gather(0, 0)
    lax.fori_loop(0, HIST, body, (), unroll=False)
    for hh in (0, 1):
        pltpu.make_async_copy(
            rt2.at[hh], out5_hbm.at[0, :, 0], osems[hh]
        ).wait()


def kernel(words, table):
    mesh = plsc.VectorSubcoreMesh(core_axis_name="c", subcore_axis_name="s")
    out5 = pl.kernel(
        _gather_body,
        out_type=jax.ShapeDtypeStruct((HIST, 8, NW, 8, 128), jnp.float32),
        mesh=mesh,
        scratch_types=[
            pltpu.VMEM((ROWS_W, HIST), jnp.int32),
            pltpu.VMEM((HIST, ROWS_W), jnp.int32),
            pltpu.VMEM((2, ROWS_W, EMBED_DIM), jnp.float32),
            pltpu.VMEM((2, 8, 8, 128), jnp.float32),
            [pltpu.SemaphoreType.DMA, pltpu.SemaphoreType.DMA],
            [pltpu.SemaphoreType.DMA, pltpu.SemaphoreType.DMA],
        ],
        compiler_params=pltpu.CompilerParams(use_tc_tiling_on_sc=False, needs_layout_passes=False),
    )(words, table)
    return out5.transpose(2, 4, 0, 1, 3).reshape(BATCH, HIST, EMBED_DIM)


# speedup vs baseline: 2.2858x; 1.0991x over previous
import jax
import jax.numpy as jnp
from jax import lax
from jax.experimental import pallas as pl
from jax.experimental.pallas import tpu as pltpu
from jax.experimental.pallas import tpu_sc as plsc

BATCH = 4096
HIST = 200
EMBED_DIM = 64
V = 1_000_000

NC = 2
NS = 16
NW = NC * NS
ROWS_W = BATCH // NW

NBLK = V // 128
KPW = NBLK // NW
NLEFT = NBLK - KPW * NW
RAG_COLS = V - NBLK * 128
NBLK = V // 128
KPW = NBLK // NW
NLEFT = NBLK - KPW * NW
RAG = V - NBLK * 128


def _iota16():
    return lax.iota(jnp.int32, 16)


def _transpose_block(src, dst2, b):
    bvec = jnp.full((16,), b, jnp.int32)
    for c in range(EMBED_DIM // 16):
        val = plsc.load_gather(src, [_iota16() + c * 16, bvec])
        p = b * EMBED_DIM + c * 16
        dst2[p // 128, pl.ds(p % 128, 16)] = val


def _untile_body(tt_hbm, flat_hbm, in_a, in_b, in3, out_a, out_b, isems, osems):
    wid = lax.axis_index("s") * NC + lax.axis_index("c")

    def transpose_block(src_ref, dst1, nb):
        def step(e0, _):
            rot = jnp.remainder(_iota16() + e0, 16)
            for e1 in range(EMBED_DIM // 16):
                ev = e1 * 16 + rot
                for b0 in range(0, nb, 16):
                    bvec = _iota16() + b0
                    val = plsc.load_gather(src_ref, [ev, bvec])
                    plsc.store_scatter(dst1, [bvec * EMBED_DIM + ev], val)
            return ()

        lax.fori_loop(0, 16, step, (), unroll=False)

    ins = (in_a, in_b)
    outs = (out_a, out_b)

    def fire_in(k, h):
        v = (k * NW + wid) * 128
        pltpu.async_copy(tt_hbm.at[:, pl.ds(v, 128)], ins[h], isems[h])

    def body(k, _):
        h = jnp.remainder(k, 2)
        for hh in (0, 1):
            @pl.when(h == hh)
            def _():
                @pl.when(k + 1 < KPW)
                def _():
                    fire_in(k + 1, 1 - hh)

                pltpu.make_async_copy(
                    tt_hbm.at[:, pl.ds(0, 128)], ins[hh], isems[hh]
                ).wait()

                @pl.when(k >= 2)
                def _():
                    pltpu.make_async_copy(
                        outs[hh], flat_hbm.at[pl.ds(0, 128 * EMBED_DIM)],
                        osems[hh],
                    ).wait()

                transpose_block(ins[hh], outs[hh], 128)
                pltpu.async_copy(
                    outs[hh],
                    flat_hbm.at[
                        pl.ds((k * NW + wid) * 128 * EMBED_DIM, 128 * EMBED_DIM)
                    ],
                    osems[hh],
                )
        return ()

    fire_in(0, 0)
    lax.fori_loop(0, KPW, body, (), unroll=False)
    for hh in (0, 1):
        pltpu.make_async_copy(
            outs[hh], flat_hbm.at[pl.ds(0, 128 * EMBED_DIM)], osems[hh]
        ).wait()

    @pl.when(wid < NLEFT)
    def _():
        v = KPW * NW + wid
        pltpu.async_copy(
            tt_hbm.at[:, pl.ds(v * 128, 128)], in_a, isems[0]
        ).wait()
        transpose_block(in_a, out_a, 128)
        pltpu.async_copy(
            out_a,
            flat_hbm.at[pl.ds(v * 128 * EMBED_DIM, 128 * EMBED_DIM)],
            osems[0],
        ).wait()

    @pl.when(wid == NLEFT)
    def _():
        pltpu.async_copy(
            tt_hbm.at[:, pl.ds(NBLK * 128, RAG)], in3, isems[1]
        ).wait()
        transpose_block(in3, out_b, RAG)
        pltpu.async_copy(
            out_b.at[pl.ds(0, RAG * EMBED_DIM)],
            flat_hbm.at[pl.ds(NBLK * 128 * EMBED_DIM, RAG * EMBED_DIM)],
            osems[1],
        ).wait()


def _gather_body(words_hbm, tbl_hbm, out5_hbm, idx_v, idx_t, rows2, rt2,
                 gsems, osems):
    wid = lax.axis_index("s") * NC + lax.axis_index("c")
    base = wid * ROWS_W
    pltpu.sync_copy(words_hbm.at[pl.ds(base, ROWS_W)], idx_v)

    def tr_idx(hh, _):
        hvec = jnp.full((16,), hh, jnp.int32)
        for c in range(ROWS_W // 16):
            val = plsc.load_gather(idx_v, [_iota16() + c * 16, hvec])
            idx_t[hh, pl.ds(c * 16, 16)] = val
        return ()

    lax.fori_loop(0, HIST, tr_idx, (), unroll=False)

    def fire_gather(i, h):
        pltpu.async_copy(tbl_hbm.at[idx_t.at[i]], rows2.at[h], gsems[h])

    def transpose_rows(h):
        def step(e0, _):
            rot = jnp.remainder(_iota16() + e0, 16)
            for e1 in range(EMBED_DIM // 16):
                ev = e1 * 16 + rot
                i1 = ev // 8
                i2 = jnp.remainder(ev, 8)
                for b0 in range(0, ROWS_W, 16):
                    bvec = _iota16() + b0
                    val = plsc.load_gather(rows2.at[h], [bvec, ev])
                    plsc.store_scatter(rt2.at[h], [i1, i2, bvec], val)
            return ()

        lax.fori_loop(0, 16, step, (), unroll=False)

    def body(i, _):
        h = jnp.remainder(i, 2)
        for hh in (0, 1):
            @pl.when(h == hh)
            def _():
                @pl.when(i + 1 < HIST)
                def _():
                    fire_gather(i + 1, 1 - hh)

                pltpu.make_async_copy(
                    tbl_hbm.at[pl.ds(0, ROWS_W)], rows2.at[hh], gsems[hh]
                ).wait()

                @pl.when(i >= 2)
                def _():
                    pltpu.make_async_copy(
                        rt2.at[hh], out5_hbm.at[0, :, 0], osems[hh]
                    ).wait()

                transpose_rows(hh)
                pltpu.async_copy(rt2.at[hh], out5_hbm.at[i, :, wid], osems[hh])
        return ()

    fire_gather(0, 0)
    lax.fori_loop(0, HIST, body, (), unroll=False)
    for hh in (0, 1):
        pltpu.make_async_copy(
            rt2.at[hh], out5_hbm.at[0, :, 0], osems[hh]
        ).wait()


def kernel(words, table):
    mesh = plsc.VectorSubcoreMesh(core_axis_name="c", subcore_axis_name="s")
    tt = jnp.transpose(table)
    flat = pl.kernel(
        _untile_body,
        out_type=jax.ShapeDtypeStruct((V * EMBED_DIM,), jnp.float32),
        mesh=mesh,
        scratch_types=[
            pltpu.VMEM((EMBED_DIM, 128), jnp.float32),
            pltpu.VMEM((EMBED_DIM, 128), jnp.float32),
            pltpu.VMEM((EMBED_DIM, RAG), jnp.float32),
            pltpu.VMEM((128 * EMBED_DIM,), jnp.float32),
            pltpu.VMEM((128 * EMBED_DIM,), jnp.float32),
            [pltpu.SemaphoreType.DMA, pltpu.SemaphoreType.DMA],
            [pltpu.SemaphoreType.DMA, pltpu.SemaphoreType.DMA],
        ],
        compiler_params=pltpu.CompilerParams(
            use_tc_tiling_on_sc=True, needs_layout_passes=False
        ),
    )(tt)
    tbl_lin = flat.reshape(V, EMBED_DIM)
    out5 = pl.kernel(
        _gather_body,
        out_type=jax.ShapeDtypeStruct((HIST, 8, NW, 8, 128), jnp.float32),
        mesh=mesh,
        scratch_types=[
            pltpu.VMEM((ROWS_W, HIST), jnp.int32),
            pltpu.VMEM((HIST, ROWS_W), jnp.int32),
            pltpu.VMEM((2, ROWS_W, EMBED_DIM), jnp.float32),
            pltpu.VMEM((2, 8, 8, 128), jnp.float32),
            [pltpu.SemaphoreType.DMA, pltpu.SemaphoreType.DMA],
            [pltpu.SemaphoreType.DMA, pltpu.SemaphoreType.DMA],
        ],
        compiler_params=pltpu.CompilerParams(use_tc_tiling_on_sc=False, needs_layout_passes=False),
    )(words, tbl_lin)
    return out5.transpose(2, 4, 0, 1, 3).reshape(BATCH, HIST, EMBED_DIM)
